# baseline (speedup 1.0000x reference)
import functools

import jax
import jax.numpy as jnp
from jax import lax
from jax.experimental import pallas as pl
from jax.experimental.pallas import tpu as pltpu
from jax.experimental.pallas import tpu_sc as plsc

F = 26
V = 100000
D = 32
B = 16384
L = 16

NC, NS = 2, 16
NW = NC * NS
BPW = B // NW

CB = 64
NCHUNK = BPW // CB
IPC = CB * F
G = IPC // 128

_mesh = plsc.VectorSubcoreMesh(core_axis_name="c", subcore_axis_name="s")


@functools.partial(
    pl.kernel,
    mesh=_mesh,
    out_type=jax.ShapeDtypeStruct((B, D), jnp.float32),
    scratch_types=[
        pltpu.VMEM((IPC,), jnp.int32),
        pltpu.VMEM((IPC,), jnp.int32),
        pltpu.VMEM((IPC, D), jnp.float32),
        pltpu.VMEM((CB, D), jnp.float32),
        pltpu.SemaphoreType.DMA,
    ],
    compiler_params=pltpu.CompilerParams(use_tc_tiling_on_sc=False),
)
def _emb_kernel(c_hbm, table_hbm, out_hbm, idx_v, offs_v, rows_v, acc_v, sem):
    wid = lax.axis_index("s") * NC + lax.axis_index("c")

    for v in range(IPC // L):
        p = v * L + lax.iota(jnp.int32, L)
        offs_v[pl.ds(v * L, L)] = (p % F) * V

    def chunk_body(ci, carry):
        cb0 = wid * BPW + ci * CB
        i0 = wid * (BPW * F) + ci * IPC

        pltpu.sync_copy(c_hbm.at[pl.ds(i0, IPC)], idx_v)
        for v in range(IPC // L):
            sl = pl.ds(v * L, L)
            idx_v[sl] = idx_v[sl] + offs_v[sl]

        copies = [
            pltpu.async_copy(
                table_hbm.at[idx_v.at[pl.ds(g * 128, 128)]],
                rows_v.at[pl.ds(g * 128, 128), :],
                sem,
            )
            for g in range(G)
        ]
        for cp in copies:
            cp.wait()

        def red_body(b, carry2):
            r0 = b * F
            for h in range(D // L):
                sl = pl.ds(h * L, L)
                s = rows_v[r0, sl]
                for f in range(1, F):
                    s = s + rows_v[r0 + f, sl]
                acc_v[b, sl] = s * jnp.float32(1.0 / F)
            return carry2

        lax.fori_loop(0, CB, red_body, 0)
        pltpu.sync_copy(acc_v, out_hbm.at[pl.ds(cb0, CB), :])
        return carry

    lax.fori_loop(0, NCHUNK, chunk_body, 0)


def kernel(c, tables):
    table = tables.reshape(F * V, D)
    c_flat = c.reshape(B * F)
    return _emb_kernel(c_flat, table)
